# Initial kernel scaffold; baseline (speedup 1.0000x reference)
import jax
import jax.numpy as jnp
from jax.experimental import pallas as pl


def kernel(hidden_states, W_conf, b_conf, W_cls, b_cls, W_reg, b_reg):
    raise NotImplementedError("write your pallas kernel here")


# speedup vs baseline: 1.0653x; 1.0653x over previous
import jax
import jax.numpy as jnp
from jax.experimental import pallas as pl

_BLOCK_M = 2048


def _heads_body(x_ref, wc_ref, bc_ref, wk_ref, bk_ref, wr_ref, br_ref,
                conf_ref, cls_ref, reg_ref):
    x = x_ref[...]
    conf_ref[...] = (
        jnp.dot(x, wc_ref[...], preferred_element_type=jnp.float32) + bc_ref[...]
    )
    cls_ref[...] = (
        jnp.dot(x, wk_ref[...], preferred_element_type=jnp.float32) + bk_ref[...]
    )
    reg_ref[...] = (
        jnp.dot(x, wr_ref[...], preferred_element_type=jnp.float32) + br_ref[...]
    )


def kernel(hidden_states, W_conf, b_conf, W_cls, b_cls, W_reg, b_reg):
    B, S, H = hidden_states.shape
    M = B * S
    na = W_conf.shape[1]
    ncls = W_cls.shape[1]
    nreg = W_reg.shape[1]
    nl = ncls // na

    x = hidden_states.reshape(M, H)
    block_m = min(_BLOCK_M, M)

    conf, cls_, reg = pl.pallas_call(
        _heads_body,
        grid=(M // block_m,),
        in_specs=[
            pl.BlockSpec((block_m, H), lambda i: (i, 0)),
            pl.BlockSpec((H, na), lambda i: (0, 0)),
            pl.BlockSpec((1, na), lambda i: (0, 0)),
            pl.BlockSpec((H, ncls), lambda i: (0, 0)),
            pl.BlockSpec((1, ncls), lambda i: (0, 0)),
            pl.BlockSpec((H, nreg), lambda i: (0, 0)),
            pl.BlockSpec((1, nreg), lambda i: (0, 0)),
        ],
        out_specs=[
            pl.BlockSpec((block_m, na), lambda i: (i, 0)),
            pl.BlockSpec((block_m, ncls), lambda i: (i, 0)),
            pl.BlockSpec((block_m, nreg), lambda i: (i, 0)),
        ],
        out_shape=[
            jax.ShapeDtypeStruct((M, na), jnp.float32),
            jax.ShapeDtypeStruct((M, ncls), jnp.float32),
            jax.ShapeDtypeStruct((M, nreg), jnp.float32),
        ],
    )(
        x,
        W_conf, b_conf.reshape(1, na),
        W_cls, b_cls.reshape(1, ncls),
        W_reg, b_reg.reshape(1, nreg),
    )

    return (
        conf.reshape(B, S, na),
        cls_.reshape(B, S, na, nl),
        reg.reshape(B, S, na, 2),
    )


# speedup vs baseline: 1.1706x; 1.0989x over previous
import functools

import jax
import jax.numpy as jnp
from jax.experimental import pallas as pl

_BLOCK_M = 2048


def _heads_body(na, ncls, nreg, x_ref, w_ref, b_ref,
                conf_ref, cls_ref, reg_ref):
    y = (
        jnp.dot(x_ref[...], w_ref[...], preferred_element_type=jnp.float32)
        + b_ref[...]
    )
    conf_ref[...] = y[:, :na]
    cls_ref[...] = y[:, na:na + ncls]
    reg_ref[...] = y[:, na + ncls:]


def kernel(hidden_states, W_conf, b_conf, W_cls, b_cls, W_reg, b_reg):
    B, S, H = hidden_states.shape
    M = B * S
    na = W_conf.shape[1]
    ncls = W_cls.shape[1]
    nreg = W_reg.shape[1]
    nl = ncls // na

    x = hidden_states.reshape(M, H)
    block_m = min(_BLOCK_M, M)
    n_all = na + ncls + nreg

    w_all = jnp.concatenate([W_conf, W_cls, W_reg], axis=1)
    b_all = jnp.concatenate([b_conf, b_cls, b_reg]).reshape(1, n_all)

    body = functools.partial(_heads_body, na, ncls, nreg)

    conf, cls_, reg = pl.pallas_call(
        body,
        grid=(M // block_m,),
        in_specs=[
            pl.BlockSpec((block_m, H), lambda i: (i, 0)),
            pl.BlockSpec((H, n_all), lambda i: (0, 0)),
            pl.BlockSpec((1, n_all), lambda i: (0, 0)),
        ],
        out_specs=[
            pl.BlockSpec((block_m, na), lambda i: (i, 0)),
            pl.BlockSpec((block_m, ncls), lambda i: (i, 0)),
            pl.BlockSpec((block_m, nreg), lambda i: (i, 0)),
        ],
        out_shape=[
            jax.ShapeDtypeStruct((M, na), jnp.float32),
            jax.ShapeDtypeStruct((M, ncls), jnp.float32),
            jax.ShapeDtypeStruct((M, nreg), jnp.float32),
        ],
    )(x, w_all, b_all)

    return (
        conf.reshape(B, S, na),
        cls_.reshape(B, S, na, nl),
        reg.reshape(B, S, na, 2),
    )


# speedup vs baseline: 1.2506x; 1.0683x over previous
import functools

import jax
import jax.numpy as jnp
from jax.experimental import pallas as pl

_BLOCK_M = 4096


def _heads_body(na, ncls, nreg, x_ref, w_ref, b_ref,
                conf_ref, cls_ref, reg_ref):
    y = (
        jnp.dot(x_ref[...], w_ref[...], preferred_element_type=jnp.float32)
        + b_ref[...]
    )
    conf_ref[...] = y[:, :na]
    cls_ref[...] = y[:, na:na + ncls]
    reg_ref[...] = y[:, na + ncls:]


def kernel(hidden_states, W_conf, b_conf, W_cls, b_cls, W_reg, b_reg):
    B, S, H = hidden_states.shape
    M = B * S
    na = W_conf.shape[1]
    ncls = W_cls.shape[1]
    nreg = W_reg.shape[1]
    nl = ncls // na

    x = hidden_states.reshape(M, H)
    block_m = min(_BLOCK_M, M)
    n_all = na + ncls + nreg

    w_all = jnp.concatenate([W_conf, W_cls, W_reg], axis=1)
    b_all = jnp.concatenate([b_conf, b_cls, b_reg]).reshape(1, n_all)

    body = functools.partial(_heads_body, na, ncls, nreg)

    conf, cls_, reg = pl.pallas_call(
        body,
        grid=(M // block_m,),
        in_specs=[
            pl.BlockSpec((block_m, H), lambda i: (i, 0)),
            pl.BlockSpec((H, n_all), lambda i: (0, 0)),
            pl.BlockSpec((1, n_all), lambda i: (0, 0)),
        ],
        out_specs=[
            pl.BlockSpec((block_m, na), lambda i: (i, 0)),
            pl.BlockSpec((block_m, ncls), lambda i: (i, 0)),
            pl.BlockSpec((block_m, nreg), lambda i: (i, 0)),
        ],
        out_shape=[
            jax.ShapeDtypeStruct((M, na), jnp.float32),
            jax.ShapeDtypeStruct((M, ncls), jnp.float32),
            jax.ShapeDtypeStruct((M, nreg), jnp.float32),
        ],
    )(x, w_all, b_all)

    return (
        conf.reshape(B, S, na),
        cls_.reshape(B, S, na, nl),
        reg.reshape(B, S, na, 2),
    )
